# baseline (speedup 1.0000x reference)
import functools
from math import sqrt

import numpy as np
import jax
import jax.numpy as jnp
from jax import lax
from jax.experimental import pallas as pl
from jax.experimental.pallas import tpu as pltpu
from jax.experimental.pallas import tpu_sc as plsc

_VOCAB = 1000000
_D = 64
_SEQ = 200
_B = 1024
_PAD_IDX = 0

_NC = 2
_NS = 16
_NW = _NC * _NS
_LANES = 16
_SCALE = float(sqrt(_D))

_VBLK = 128
_NPAIR = _VOCAB // 2
_NBLK_FULL = _VOCAB // _VBLK
_BLK_REM = _VOCAB - _NBLK_FULL * _VBLK
_BLK_PER_W = _NBLK_FULL // _NW
_BLK_EXTRA = _NBLK_FULL % _NW

_SUB = 128
_NSUB = _B // _SUB
_SLAB_PER_W = _SEQ // _NW
_SLAB_EXTRA = _SEQ % _NW


def _pe_table():
    pos = np.arange(_SEQ, dtype=np.float32)[:, None]
    i = np.arange(_D, dtype=np.float32)[None, :]
    angle_rates = 1.0 / np.power(10000.0, (2.0 * np.floor(i / 2.0)) / _D)
    angles = pos * angle_rates
    pe = np.zeros((_SEQ, _D), dtype=np.float32)
    pe[:, 0::2] = np.sin(angles[:, 0::2])
    pe[:, 1::2] = np.cos(angles[:, 1::2])
    return pe


_PE_CONST = _pe_table().reshape(-1)


def _wid():
    return lax.axis_index("s") * _NC + lax.axis_index("c")


def _stage_body(tt_hbm, tail_hbm, stg_hbm, src0, src1, dst0, dst1,
                gs0, gs1, ss0, ss1):
    srcs = (src0, src1)
    dsts = (dst0, dst1)
    gsem = (gs0, gs1)
    ssem = (ss0, ss1)
    w = _wid()
    ntrip = jnp.where(w < _BLK_EXTRA, _BLK_PER_W + 1, _BLK_PER_W)

    iota = lax.iota(jnp.int32, _LANES)

    def blk_of(t):
        return w + t * _NW

    def load_start(t, r):
        v0 = blk_of(t) * _VBLK
        return pltpu.async_copy(
            tt_hbm.at[:, pl.ds(v0, _VBLK)], srcs[r], gsem[r]
        )

    def load_wait(r):
        pltpu.make_async_copy(
            tt_hbm.at[:, pl.ds(0, _VBLK)], srcs[r], gsem[r]
        ).wait()

    def store_start(t, r):
        p0 = blk_of(t) * (_VBLK // 2)
        return pltpu.async_copy(
            dsts[r], stg_hbm.at[pl.ds(p0, _VBLK // 2)], ssem[r]
        )

    def store_wait(r):
        pltpu.make_async_copy(
            stg_hbm.at[pl.ds(0, _VBLK // 2)], dsts[r], ssem[r]
        ).wait()

    def transpose_blk(r, ncols):
        src = srcs[r]
        dst = dsts[r]
        npair = ncols // 2

        @plsc.parallel_loop(0, npair, unroll=2)
        def _(p):
            for h in range(2):
                col = jnp.broadcast_to(2 * p + h, (_LANES,))
                for k in range(_D // _LANES):
                    vals = plsc.load_gather(src, [k * _LANES + iota, col])
                    dst[p, pl.ds(h * _D + k * _LANES, _LANES)] = vals

    load_start(jnp.int32(0), 0)

    def trip(t, carry):
        for r in range(2):
            tt = t * 2 + r

            @pl.when(tt < ntrip)
            def _():
                @pl.when(tt + 1 < ntrip)
                def _():
                    @pl.when(tt >= 1)
                    def _():
                        store_wait(1 - r)

                    load_start(tt + 1, 1 - r)

                load_wait(r)
                transpose_blk(r, _VBLK)
                store_start(tt, r)
        return carry

    lax.fori_loop(0, (_BLK_PER_W + 2) // 2, trip, 0)

    @pl.when(ntrip >= 1)
    def _():
        @pl.when((ntrip % 2) == jnp.int32(1))
        def _():
            store_wait(0)

        @pl.when((ntrip % 2) == jnp.int32(0))
        def _():
            store_wait(1)

    @pl.when(ntrip >= 2)
    def _():
        @pl.when((ntrip % 2) == jnp.int32(1))
        def _():
            store_wait(1)

        @pl.when((ntrip % 2) == jnp.int32(0))
        def _():
            store_wait(0)

    @pl.when(w == _NW - 1)
    def _():
        pltpu.sync_copy(tail_hbm, dst0.at[pl.ds(0, _BLK_REM // 2)])
        pltpu.async_copy(
            dst0.at[pl.ds(0, _BLK_REM // 2)],
            stg_hbm.at[pl.ds(_NPAIR - _BLK_REM // 2, _BLK_REM // 2)],
            ss0,
        ).wait()


def _gather_body(stg_hbm, xt_hbm, pe_hbm, out_hbm,
                 idx_v, pidx_v, pe_v, pes_v, out_v, pr0, pr1, gs0, gs1, osem):
    prs = (pr0, pr1)
    gsem = (gs0, gs1)
    w = _wid()
    nslab = jnp.where(w < _SLAB_EXTRA, _SLAB_PER_W + 1, _SLAB_PER_W)

    iota = lax.iota(jnp.int32, _LANES)
    pltpu.sync_copy(pe_hbm, pe_v)

    def gather_start(ss, r):
        return pltpu.async_copy(
            stg_hbm.at[pidx_v.at[pl.ds(ss * _SUB, _SUB)]], prs[r], gsem[r]
        )

    def gather_wait(r):
        pltpu.make_async_copy(
            stg_hbm.at[pidx_v.at[pl.ds(0, _SUB)]], prs[r], gsem[r]
        ).wait()

    def slab_body(si, carry):
        s = w + si * _NW
        pltpu.sync_copy(xt_hbm.at[s], idx_v)

        @plsc.parallel_loop(0, _B // _LANES, unroll=4)
        def _(g):
            sl = pl.ds(g * _LANES, _LANES)
            pidx_v[sl] = lax.shift_right_logical(idx_v[sl], 1)

        @plsc.parallel_loop(0, _D, unroll=2)
        def _(d):
            pes_v[d, pl.ds(0, _LANES)] = plsc.load_gather(
                pe_v, [jnp.broadcast_to(s * _D + d, (_LANES,))]
            )

        gather_start(jnp.int32(0), 0)

        def sub_body(ss, carry2):
            for r in range(2):
                sscur = ss * 2 + r

                @pl.when(sscur < _NSUB)
                def _():
                    @pl.when(sscur + 1 < _NSUB)
                    def _():
                        gather_start(sscur + 1, 1 - r)

                    gather_wait(r)
                    b0 = sscur * _SUB
                    pairs = prs[r]

                    def group_body(g, carry3):
                        bg = b0 + g * _LANES
                        idxv = idx_v[pl.ds(bg, _LANES)]
                        pad = idxv == _PAD_IDX
                        halfoff = (idxv & 1) * _D
                        rowvec = g * _LANES + iota

                        def d_body(d, carry4):
                            vals = plsc.load_gather(
                                pairs, [rowvec, halfoff + d]
                            )
                            pevec = pes_v[d, pl.ds(0, _LANES)]
                            res = vals * _SCALE + pevec
                            out_v[d, pl.ds(bg, _LANES)] = jnp.where(
                                pad, pevec, res
                            )
                            return carry4

                        lax.fori_loop(0, _D, d_body, 0, unroll=4)
                        return carry3

                    lax.fori_loop(0, _SUB // _LANES, group_body, 0)
            return carry2

        lax.fori_loop(0, (_NSUB + 1) // 2, sub_body, 0)

        pltpu.async_copy(out_v, out_hbm.at[s], osem).wait()
        return carry

    lax.fori_loop(0, nslab, slab_body, 0)


_mesh = plsc.VectorSubcoreMesh(core_axis_name="c", subcore_axis_name="s")
_params = pltpu.CompilerParams(
    needs_layout_passes=False, use_tc_tiling_on_sc=True
)

_stage_call = functools.partial(
    pl.kernel,
    mesh=_mesh,
    out_type=jax.ShapeDtypeStruct((_NPAIR, 2 * _D), jnp.float32),
    scratch_types=[
        pltpu.VMEM((_D, _VBLK), jnp.float32),
        pltpu.VMEM((_D, _VBLK), jnp.float32),
        pltpu.VMEM((_VBLK // 2, 2 * _D), jnp.float32),
        pltpu.VMEM((_VBLK // 2, 2 * _D), jnp.float32),
    ]
    + [pltpu.SemaphoreType.DMA for _ in range(4)],
    compiler_params=_params,
)(_stage_body)

_gather_call = functools.partial(
    pl.kernel,
    mesh=_mesh,
    out_type=jax.ShapeDtypeStruct((_SEQ, _D, _B), jnp.float32),
    scratch_types=[
        pltpu.VMEM((_B,), jnp.int32),
        pltpu.VMEM((_B,), jnp.int32),
        pltpu.VMEM((_SEQ * _D,), jnp.float32),
        pltpu.VMEM((_D, _LANES), jnp.float32),
        pltpu.VMEM((_D, _B), jnp.float32),
        pltpu.VMEM((_SUB, 2 * _D), jnp.float32),
        pltpu.VMEM((_SUB, 2 * _D), jnp.float32),
    ]
    + [pltpu.SemaphoreType.DMA for _ in range(3)],
    compiler_params=_params,
)(_gather_body)


@jax.jit
def kernel(x, table):
    xt = x.astype(jnp.int32).T
    tt = table.T
    tail = table[_NBLK_FULL * _VBLK:].reshape(_BLK_REM // 2, 2 * _D)
    stg = _stage_call(tt, tail)
    out = _gather_call(stg, xt, _PE_CONST)
    return out.transpose(2, 0, 1)


# speedup vs baseline: 3.4841x; 3.4841x over previous
import functools
from math import sqrt

import numpy as np
import jax
import jax.numpy as jnp
from jax import lax
from jax.experimental import pallas as pl
from jax.experimental.pallas import tpu as pltpu
from jax.experimental.pallas import tpu_sc as plsc

_VOCAB = 1000000
_D = 64
_SEQ = 200
_B = 1024
_PAD_IDX = 0

_NC = 2
_NS = 16
_NW = _NC * _NS
_LANES = 16
_SCALE = float(sqrt(_D))

_VBLK = 128
_NPAIR = _VOCAB // 2
_NBLK_FULL = _VOCAB // _VBLK
_BLK_REM = _VOCAB - _NBLK_FULL * _VBLK
_BLK_PER_W = _NBLK_FULL // _NW
_BLK_EXTRA = _NBLK_FULL % _NW

_SUB = 128
_NSUB = _B // _SUB
_SLAB_PER_W = _SEQ // _NW
_SLAB_EXTRA = _SEQ % _NW


def _pe_table():
    pos = np.arange(_SEQ, dtype=np.float32)[:, None]
    i = np.arange(_D, dtype=np.float32)[None, :]
    angle_rates = 1.0 / np.power(10000.0, (2.0 * np.floor(i / 2.0)) / _D)
    angles = pos * angle_rates
    pe = np.zeros((_SEQ, _D), dtype=np.float32)
    pe[:, 0::2] = np.sin(angles[:, 0::2])
    pe[:, 1::2] = np.cos(angles[:, 1::2])
    return pe


_PE_CONST = _pe_table().reshape(-1)


def _wid():
    return lax.axis_index("s") * _NC + lax.axis_index("c")


def _stage_body(tt_hbm, tail_hbm, stg_hbm, src0, src1, dst0, dst1,
                gs0, gs1, ss0, ss1):
    srcs = (src0, src1)
    dsts = (dst0, dst1)
    gsem = (gs0, gs1)
    ssem = (ss0, ss1)
    w = _wid()
    ntrip = jnp.where(w < _BLK_EXTRA, _BLK_PER_W + 1, _BLK_PER_W)

    iota = lax.iota(jnp.int32, _LANES)

    def blk_of(t):
        return w + t * _NW

    def load_start(t, r):
        v0 = blk_of(t) * _VBLK
        return pltpu.async_copy(
            tt_hbm.at[:, pl.ds(v0, _VBLK)], srcs[r], gsem[r]
        )

    def load_wait(r):
        pltpu.make_async_copy(
            tt_hbm.at[:, pl.ds(0, _VBLK)], srcs[r], gsem[r]
        ).wait()

    def store_start(t, r):
        p0 = blk_of(t) * (_VBLK // 2)
        return pltpu.async_copy(
            dsts[r], stg_hbm.at[pl.ds(p0, _VBLK // 2)], ssem[r]
        )

    def store_wait(r):
        pltpu.make_async_copy(
            stg_hbm.at[pl.ds(0, _VBLK // 2)], dsts[r], ssem[r]
        ).wait()

    def transpose_blk(r, ncols):
        src = srcs[r]
        dst = dsts[r]

        @plsc.parallel_loop(0, ncols, unroll=2)
        def _(c0):
            cvec = (c0 + iota) & (ncols - 1)
            rowvec = lax.shift_right_logical(cvec, 1)
            halfvec = (cvec & 1) * _D
            for kd in range(_D // _LANES):
                dvec = kd * _LANES + iota
                vals = plsc.load_gather(src, [dvec, cvec])
                plsc.store_scatter(dst, [rowvec, halfvec + dvec], vals)

    load_start(jnp.int32(0), 0)

    def trip(t, carry):
        for r in range(2):
            tt = t * 2 + r

            @pl.when(tt < ntrip)
            def _():
                @pl.when(tt + 1 < ntrip)
                def _():
                    @pl.when(tt >= 1)
                    def _():
                        store_wait(1 - r)

                    load_start(tt + 1, 1 - r)

                load_wait(r)
                transpose_blk(r, _VBLK)
                store_start(tt, r)
        return carry

    lax.fori_loop(0, (_BLK_PER_W + 2) // 2, trip, 0)

    @pl.when(ntrip >= 1)
    def _():
        @pl.when((ntrip % 2) == jnp.int32(1))
        def _():
            store_wait(0)

        @pl.when((ntrip % 2) == jnp.int32(0))
        def _():
            store_wait(1)

    @pl.when(ntrip >= 2)
    def _():
        @pl.when((ntrip % 2) == jnp.int32(1))
        def _():
            store_wait(1)

        @pl.when((ntrip % 2) == jnp.int32(0))
        def _():
            store_wait(0)

    @pl.when(w == _NW - 1)
    def _():
        pltpu.sync_copy(tail_hbm, dst0.at[pl.ds(0, _BLK_REM // 2)])
        pltpu.async_copy(
            dst0.at[pl.ds(0, _BLK_REM // 2)],
            stg_hbm.at[pl.ds(_NPAIR - _BLK_REM // 2, _BLK_REM // 2)],
            ss0,
        ).wait()


def _gather_body(stg_hbm, xt_hbm, pe_hbm, out_hbm,
                 idx_v, pidx_v, pe_v, out_v, pr0, pr1, gs0, gs1, osem):
    prs = (pr0, pr1)
    gsem = (gs0, gs1)
    w = _wid()
    nslab = jnp.where(w < _SLAB_EXTRA, _SLAB_PER_W + 1, _SLAB_PER_W)

    iota = lax.iota(jnp.int32, _LANES)
    pltpu.sync_copy(pe_hbm, pe_v)

    def gather_start(ss, r):
        return pltpu.async_copy(
            stg_hbm.at[pidx_v.at[pl.ds(ss * _SUB, _SUB)]], prs[r], gsem[r]
        )

    def gather_wait(r):
        pltpu.make_async_copy(
            stg_hbm.at[pidx_v.at[pl.ds(0, _SUB)]], prs[r], gsem[r]
        ).wait()

    def slab_body(si, carry):
        s = w + si * _NW
        pltpu.sync_copy(xt_hbm.at[s], idx_v)

        @plsc.parallel_loop(0, _B // _LANES, unroll=4)
        def _(g):
            sl = pl.ds(g * _LANES, _LANES)
            pidx_v[sl] = lax.shift_right_logical(idx_v[sl], 1)

        gather_start(jnp.int32(0), 0)
        pe0 = s * _D

        def sub_body(ss, carry2):
            for r in range(2):
                sscur = ss * 2 + r

                @pl.when(sscur < _NSUB)
                def _():
                    @pl.when(sscur + 1 < _NSUB)
                    def _():
                        gather_start(sscur + 1, 1 - r)

                    gather_wait(r)
                    b0 = sscur * _SUB
                    pairs = prs[r]

                    def group_body(g, carry3):
                        bg = b0 + g * _LANES
                        idxv = idx_v[pl.ds(bg, _LANES)]
                        pad = idxv == _PAD_IDX
                        halfoff = (idxv & 1) * _D
                        rowvec = g * _LANES + iota
                        bvec = bg + iota

                        @plsc.parallel_loop(0, _D, unroll=4)
                        def _(d0):
                            dvec = (d0 + iota) & (_D - 1)
                            vals = plsc.load_gather(
                                pairs, [rowvec, halfoff + dvec]
                            )
                            pevec = plsc.load_gather(pe_v, [pe0 + dvec])
                            res = vals * _SCALE + pevec
                            plsc.store_scatter(
                                out_v, [dvec, bvec],
                                jnp.where(pad, pevec, res),
                            )

                        return carry3

                    lax.fori_loop(0, _SUB // _LANES, group_body, 0)
            return carry2

        lax.fori_loop(0, (_NSUB + 1) // 2, sub_body, 0)

        pltpu.async_copy(out_v, out_hbm.at[s], osem).wait()
        return carry

    lax.fori_loop(0, nslab, slab_body, 0)


_mesh = plsc.VectorSubcoreMesh(core_axis_name="c", subcore_axis_name="s")
_params = pltpu.CompilerParams(
    needs_layout_passes=False, use_tc_tiling_on_sc=True
)

_stage_call = functools.partial(
    pl.kernel,
    mesh=_mesh,
    out_type=jax.ShapeDtypeStruct((_NPAIR, 2 * _D), jnp.float32),
    scratch_types=[
        pltpu.VMEM((_D, _VBLK), jnp.float32),
        pltpu.VMEM((_D, _VBLK), jnp.float32),
        pltpu.VMEM((_VBLK // 2, 2 * _D), jnp.float32),
        pltpu.VMEM((_VBLK // 2, 2 * _D), jnp.float32),
    ]
    + [pltpu.SemaphoreType.DMA for _ in range(4)],
    compiler_params=_params,
)(_stage_body)

_gather_call = functools.partial(
    pl.kernel,
    mesh=_mesh,
    out_type=jax.ShapeDtypeStruct((_SEQ, _D, _B), jnp.float32),
    scratch_types=[
        pltpu.VMEM((_B,), jnp.int32),
        pltpu.VMEM((_B,), jnp.int32),
        pltpu.VMEM((_SEQ * _D,), jnp.float32),
        pltpu.VMEM((_D, _B), jnp.float32),
        pltpu.VMEM((_SUB, 2 * _D), jnp.float32),
        pltpu.VMEM((_SUB, 2 * _D), jnp.float32),
    ]
    + [pltpu.SemaphoreType.DMA for _ in range(3)],
    compiler_params=_params,
)(_gather_body)


@jax.jit
def kernel(x, table):
    xt = x.astype(jnp.int32).T
    tt = table.T
    tail = table[_NBLK_FULL * _VBLK:].reshape(_BLK_REM // 2, 2 * _D)
    stg = _stage_call(tt, tail)
    out = _gather_call(stg, xt, _PE_CONST)
    return out.transpose(2, 0, 1)


# speedup vs baseline: 4.0167x; 1.1529x over previous
import functools
from math import sqrt

import numpy as np
import jax
import jax.numpy as jnp
from jax import lax
from jax.experimental import pallas as pl
from jax.experimental.pallas import tpu as pltpu
from jax.experimental.pallas import tpu_sc as plsc

_VOCAB = 1000000
_D = 64
_SEQ = 200
_B = 1024
_PAD_IDX = 0

_NC = 2
_NS = 16
_NW = _NC * _NS
_LANES = 16
_SCALE = float(sqrt(_D))

_VBLK = 256
_NPAIR = _VOCAB // 2
_NBLK_FULL = _VOCAB // _VBLK
_BLK_REM = _VOCAB - _NBLK_FULL * _VBLK
_BLK_PER_W = _NBLK_FULL // _NW
_BLK_EXTRA = _NBLK_FULL % _NW

_SUB = 128
_NSUB = _B // _SUB
_SLAB_PER_W = _SEQ // _NW
_SLAB_EXTRA = _SEQ % _NW


def _pe_table():
    pos = np.arange(_SEQ, dtype=np.float32)[:, None]
    i = np.arange(_D, dtype=np.float32)[None, :]
    angle_rates = 1.0 / np.power(10000.0, (2.0 * np.floor(i / 2.0)) / _D)
    angles = pos * angle_rates
    pe = np.zeros((_SEQ, _D), dtype=np.float32)
    pe[:, 0::2] = np.sin(angles[:, 0::2])
    pe[:, 1::2] = np.cos(angles[:, 1::2])
    return pe


_PE_CONST = _pe_table().reshape(-1)


def _wid():
    return lax.axis_index("s") * _NC + lax.axis_index("c")


def _stage_body(tt_hbm, tail_hbm, stg_hbm, src0, src1, dst0, dst1,
                gs0, gs1, ss0, ss1):
    srcs = (src0, src1)
    dsts = (dst0, dst1)
    gsem = (gs0, gs1)
    ssem = (ss0, ss1)
    w = _wid()
    ntrip = jnp.where(w < _BLK_EXTRA, _BLK_PER_W + 1, _BLK_PER_W)

    iota = lax.iota(jnp.int32, _LANES)

    def blk_of(t):
        return w + t * _NW

    def load_start(t, r):
        v0 = blk_of(t) * _VBLK
        return pltpu.async_copy(
            tt_hbm.at[:, pl.ds(v0, _VBLK)], srcs[r], gsem[r]
        )

    def load_wait(r):
        pltpu.make_async_copy(
            tt_hbm.at[:, pl.ds(0, _VBLK)], srcs[r], gsem[r]
        ).wait()

    def store_start(t, r):
        p0 = blk_of(t) * (_VBLK // 2)
        return pltpu.async_copy(
            dsts[r], stg_hbm.at[pl.ds(p0, _VBLK // 2)], ssem[r]
        )

    def store_wait(r):
        pltpu.make_async_copy(
            stg_hbm.at[pl.ds(0, _VBLK // 2)], dsts[r], ssem[r]
        ).wait()

    def transpose_blk(r, ncols):
        src = srcs[r]
        dst = dsts[r]

        @plsc.parallel_loop(0, ncols, unroll=2)
        def _(c0):
            cvec = (c0 + iota) & (ncols - 1)
            rowvec = lax.shift_right_logical(cvec, 1)
            halfvec = (cvec & 1) * _D
            for kd in range(_D // _LANES):
                dvec = kd * _LANES + iota
                vals = plsc.load_gather(src, [dvec, cvec])
                plsc.store_scatter(dst, [rowvec, halfvec + dvec], vals)

    load_start(jnp.int32(0), 0)

    def trip(t, carry):
        for r in range(2):
            tt = t * 2 + r

            @pl.when(tt < ntrip)
            def _():
                @pl.when(tt + 1 < ntrip)
                def _():
                    @pl.when(tt >= 1)
                    def _():
                        store_wait(1 - r)

                    load_start(tt + 1, 1 - r)

                load_wait(r)
                transpose_blk(r, _VBLK)
                store_start(tt, r)
        return carry

    lax.fori_loop(0, (_BLK_PER_W + 2) // 2, trip, 0)

    @pl.when(ntrip >= 1)
    def _():
        @pl.when((ntrip % 2) == jnp.int32(1))
        def _():
            store_wait(0)

        @pl.when((ntrip % 2) == jnp.int32(0))
        def _():
            store_wait(1)

    @pl.when(ntrip >= 2)
    def _():
        @pl.when((ntrip % 2) == jnp.int32(1))
        def _():
            store_wait(1)

        @pl.when((ntrip % 2) == jnp.int32(0))
        def _():
            store_wait(0)

    @pl.when(w == _NW - 1)
    def _():
        pltpu.sync_copy(tail_hbm, dst0.at[pl.ds(0, _BLK_REM // 2)])
        pltpu.async_copy(
            dst0.at[pl.ds(0, _BLK_REM // 2)],
            stg_hbm.at[pl.ds(_NPAIR - _BLK_REM // 2, _BLK_REM // 2)],
            ss0,
        ).wait()


def _gather_body(stg_hbm, xt_hbm, pe_hbm, out_hbm,
                 idx_v, pidx_v, pe_v, out_v, pr0, pr1, gs0, gs1, osem):
    prs = (pr0, pr1)
    gsem = (gs0, gs1)
    w = _wid()
    nslab = jnp.where(w < _SLAB_EXTRA, _SLAB_PER_W + 1, _SLAB_PER_W)

    iota = lax.iota(jnp.int32, _LANES)
    pltpu.sync_copy(pe_hbm, pe_v)

    def gather_start(ss, r):
        return pltpu.async_copy(
            stg_hbm.at[pidx_v.at[pl.ds(ss * _SUB, _SUB)]], prs[r], gsem[r]
        )

    def gather_wait(r):
        pltpu.make_async_copy(
            stg_hbm.at[pidx_v.at[pl.ds(0, _SUB)]], prs[r], gsem[r]
        ).wait()

    def slab_body(si, carry):
        s = w + si * _NW
        pltpu.sync_copy(xt_hbm.at[s], idx_v)

        @plsc.parallel_loop(0, _B // _LANES, unroll=4)
        def _(g):
            sl = pl.ds(g * _LANES, _LANES)
            pidx_v[sl] = lax.shift_right_logical(idx_v[sl], 1)

        gather_start(jnp.int32(0), 0)
        pe0 = s * _D

        def sub_body(ss, carry2):
            for r in range(2):
                sscur = ss * 2 + r

                @pl.when(sscur < _NSUB)
                def _():
                    @pl.when(sscur + 1 < _NSUB)
                    def _():
                        gather_start(sscur + 1, 1 - r)

                    gather_wait(r)
                    b0 = sscur * _SUB
                    pairs = prs[r]

                    def group_body(g, carry3):
                        bg = b0 + g * _LANES
                        idxv = idx_v[pl.ds(bg, _LANES)]
                        scalev = jnp.where(
                            idxv == _PAD_IDX,
                            jnp.float32(0.0),
                            jnp.float32(_SCALE),
                        )
                        halfoff = (idxv & 1) * _D
                        rowvec = g * _LANES + iota
                        bvec = bg + iota

                        @plsc.parallel_loop(0, _D, unroll=4)
                        def _(d0):
                            dvec = (d0 + iota) & (_D - 1)
                            vals = plsc.load_gather(
                                pairs, [rowvec, halfoff + dvec]
                            )
                            pevec = plsc.load_gather(pe_v, [pe0 + dvec])
                            plsc.store_scatter(
                                out_v, [dvec, bvec], vals * scalev + pevec
                            )

                        return carry3

                    lax.fori_loop(0, _SUB // _LANES, group_body, 0)
            return carry2

        lax.fori_loop(0, (_NSUB + 1) // 2, sub_body, 0)

        pltpu.async_copy(out_v, out_hbm.at[s], osem).wait()
        return carry

    lax.fori_loop(0, nslab, slab_body, 0)


_mesh = plsc.VectorSubcoreMesh(core_axis_name="c", subcore_axis_name="s")
_params = pltpu.CompilerParams(
    needs_layout_passes=False, use_tc_tiling_on_sc=True
)

_stage_call = functools.partial(
    pl.kernel,
    mesh=_mesh,
    out_type=jax.ShapeDtypeStruct((_NPAIR, 2 * _D), jnp.float32),
    scratch_types=[
        pltpu.VMEM((_D, _VBLK), jnp.float32),
        pltpu.VMEM((_D, _VBLK), jnp.float32),
        pltpu.VMEM((_VBLK // 2, 2 * _D), jnp.float32),
        pltpu.VMEM((_VBLK // 2, 2 * _D), jnp.float32),
    ]
    + [pltpu.SemaphoreType.DMA for _ in range(4)],
    compiler_params=_params,
)(_stage_body)

_gather_call = functools.partial(
    pl.kernel,
    mesh=_mesh,
    out_type=jax.ShapeDtypeStruct((_SEQ, _D, _B), jnp.float32),
    scratch_types=[
        pltpu.VMEM((_B,), jnp.int32),
        pltpu.VMEM((_B,), jnp.int32),
        pltpu.VMEM((_SEQ * _D,), jnp.float32),
        pltpu.VMEM((_D, _B), jnp.float32),
        pltpu.VMEM((_SUB, 2 * _D), jnp.float32),
        pltpu.VMEM((_SUB, 2 * _D), jnp.float32),
    ]
    + [pltpu.SemaphoreType.DMA for _ in range(3)],
    compiler_params=_params,
)(_gather_body)


@jax.jit
def kernel(x, table):
    xt = x.astype(jnp.int32).T
    tt = table.T
    tail = table[_NBLK_FULL * _VBLK:].reshape(_BLK_REM // 2, 2 * _D)
    stg = _stage_call(tt, tail)
    out = _gather_call(stg, xt, _PE_CONST)
    return out.transpose(2, 0, 1)


# speedup vs baseline: 4.0762x; 1.0148x over previous
import functools
from math import sqrt

import numpy as np
import jax
import jax.numpy as jnp
from jax import lax
from jax.experimental import pallas as pl
from jax.experimental.pallas import tpu as pltpu
from jax.experimental.pallas import tpu_sc as plsc

_VOCAB = 1000000
_D = 64
_SEQ = 200
_B = 1024
_PAD_IDX = 0

_NC = 2
_NS = 16
_NW = _NC * _NS
_LANES = 16
_SCALE = float(sqrt(_D))

_VBLK = 256
_NPAIR = _VOCAB // 2
_NBLK_FULL = _VOCAB // _VBLK
_BLK_REM = _VOCAB - _NBLK_FULL * _VBLK
_BLK_PER_W = _NBLK_FULL // _NW
_BLK_EXTRA = _NBLK_FULL % _NW

_SUB = 128
_NSUB = _B // _SUB
_SLAB_PER_W = _SEQ // _NW
_SLAB_EXTRA = _SEQ % _NW


def _pe_table():
    pos = np.arange(_SEQ, dtype=np.float32)[:, None]
    i = np.arange(_D, dtype=np.float32)[None, :]
    angle_rates = 1.0 / np.power(10000.0, (2.0 * np.floor(i / 2.0)) / _D)
    angles = pos * angle_rates
    pe = np.zeros((_SEQ, _D), dtype=np.float32)
    pe[:, 0::2] = np.sin(angles[:, 0::2])
    pe[:, 1::2] = np.cos(angles[:, 1::2])
    return pe


_PE_CONST = _pe_table().reshape(-1)


def _wid():
    return lax.axis_index("s") * _NC + lax.axis_index("c")


def _stage_body(tt_hbm, tail_hbm, stg_hbm, src0, src1, dst0, dst1,
                gs0, gs1, ss0, ss1):
    srcs = (src0, src1)
    dsts = (dst0, dst1)
    gsem = (gs0, gs1)
    ssem = (ss0, ss1)
    w = _wid()
    ntrip = jnp.where(w < _BLK_EXTRA, _BLK_PER_W + 1, _BLK_PER_W)

    iota = lax.iota(jnp.int32, _LANES)

    def blk_of(t):
        return w + t * _NW

    def load_start(t, r):
        v0 = blk_of(t) * _VBLK
        return pltpu.async_copy(
            tt_hbm.at[:, pl.ds(v0, _VBLK)], srcs[r], gsem[r]
        )

    def load_wait(r):
        pltpu.make_async_copy(
            tt_hbm.at[:, pl.ds(0, _VBLK)], srcs[r], gsem[r]
        ).wait()

    def store_start(t, r):
        p0 = blk_of(t) * (_VBLK // 2)
        return pltpu.async_copy(
            dsts[r], stg_hbm.at[pl.ds(p0, _VBLK // 2)], ssem[r]
        )

    def store_wait(r):
        pltpu.make_async_copy(
            stg_hbm.at[pl.ds(0, _VBLK // 2)], dsts[r], ssem[r]
        ).wait()

    def transpose_blk(r, ncols):
        src = srcs[r]
        dst = dsts[r]

        @plsc.parallel_loop(0, ncols, unroll=4)
        def _(c0):
            cvec = (c0 + iota) & (ncols - 1)
            rowvec = lax.shift_right_logical(cvec, 1)
            halfvec = (cvec & 1) * _D
            for kd in range(_D // _LANES):
                dvec = kd * _LANES + iota
                vals = plsc.load_gather(src, [dvec, cvec])
                plsc.store_scatter(dst, [rowvec, halfvec + dvec], vals)

    load_start(jnp.int32(0), 0)

    def trip(t, carry):
        for r in range(2):
            tt = t * 2 + r

            @pl.when(tt < ntrip)
            def _():
                @pl.when(tt + 1 < ntrip)
                def _():
                    @pl.when(tt >= 1)
                    def _():
                        store_wait(1 - r)

                    load_start(tt + 1, 1 - r)

                load_wait(r)
                transpose_blk(r, _VBLK)
                store_start(tt, r)
        return carry

    lax.fori_loop(0, (_BLK_PER_W + 2) // 2, trip, 0)

    @pl.when(ntrip >= 1)
    def _():
        @pl.when((ntrip % 2) == jnp.int32(1))
        def _():
            store_wait(0)

        @pl.when((ntrip % 2) == jnp.int32(0))
        def _():
            store_wait(1)

    @pl.when(ntrip >= 2)
    def _():
        @pl.when((ntrip % 2) == jnp.int32(1))
        def _():
            store_wait(1)

        @pl.when((ntrip % 2) == jnp.int32(0))
        def _():
            store_wait(0)

    @pl.when(w == _NW - 1)
    def _():
        pltpu.sync_copy(tail_hbm, dst0.at[pl.ds(0, _BLK_REM // 2)])
        pltpu.async_copy(
            dst0.at[pl.ds(0, _BLK_REM // 2)],
            stg_hbm.at[pl.ds(_NPAIR - _BLK_REM // 2, _BLK_REM // 2)],
            ss0,
        ).wait()


def _gather_body(stg_hbm, xt_hbm, pe_hbm, out_hbm,
                 idx_v, pidx_v, pe_v, out_v, pr0, pr1, gs0, gs1, osem):
    prs = (pr0, pr1)
    gsem = (gs0, gs1)
    w = _wid()
    nslab = jnp.where(w < _SLAB_EXTRA, _SLAB_PER_W + 1, _SLAB_PER_W)

    iota = lax.iota(jnp.int32, _LANES)
    pltpu.sync_copy(pe_hbm, pe_v)

    def gather_start(ss, r):
        return pltpu.async_copy(
            stg_hbm.at[pidx_v.at[pl.ds(ss * _SUB, _SUB)]], prs[r], gsem[r]
        )

    def gather_wait(r):
        pltpu.make_async_copy(
            stg_hbm.at[pidx_v.at[pl.ds(0, _SUB)]], prs[r], gsem[r]
        ).wait()

    def slab_body(si, carry):
        s = w + si * _NW
        pltpu.sync_copy(xt_hbm.at[s], idx_v)

        @plsc.parallel_loop(0, _B // _LANES, unroll=4)
        def _(g):
            sl = pl.ds(g * _LANES, _LANES)
            pidx_v[sl] = lax.shift_right_logical(idx_v[sl], 1)

        gather_start(jnp.int32(0), 0)
        pe0 = s * _D

        def sub_body(ss, carry2):
            for r in range(2):
                sscur = ss * 2 + r

                @pl.when(sscur < _NSUB)
                def _():
                    @pl.when(sscur + 1 < _NSUB)
                    def _():
                        gather_start(sscur + 1, 1 - r)

                    gather_wait(r)
                    b0 = sscur * _SUB
                    pairs = prs[r]

                    def group_body(g, carry3):
                        bg = b0 + g * _LANES
                        idxv = idx_v[pl.ds(bg, _LANES)]
                        scalev = jnp.where(
                            idxv == _PAD_IDX,
                            jnp.float32(0.0),
                            jnp.float32(_SCALE),
                        )
                        halfoff = (idxv & 1) * _D
                        rowvec = g * _LANES + iota
                        bvec = bg + iota

                        @plsc.parallel_loop(0, _D, unroll=8)
                        def _(d0):
                            dvec = (d0 + iota) & (_D - 1)
                            vals = plsc.load_gather(
                                pairs, [rowvec, halfoff + dvec]
                            )
                            pevec = plsc.load_gather(pe_v, [pe0 + dvec])
                            plsc.store_scatter(
                                out_v, [dvec, bvec], vals * scalev + pevec
                            )

                        return carry3

                    lax.fori_loop(0, _SUB // _LANES, group_body, 0)
            return carry2

        lax.fori_loop(0, (_NSUB + 1) // 2, sub_body, 0)

        pltpu.async_copy(out_v, out_hbm.at[s], osem).wait()
        return carry

    lax.fori_loop(0, nslab, slab_body, 0)


_mesh = plsc.VectorSubcoreMesh(core_axis_name="c", subcore_axis_name="s")
_params = pltpu.CompilerParams(
    needs_layout_passes=False, use_tc_tiling_on_sc=True
)

_stage_call = functools.partial(
    pl.kernel,
    mesh=_mesh,
    out_type=jax.ShapeDtypeStruct((_NPAIR, 2 * _D), jnp.float32),
    scratch_types=[
        pltpu.VMEM((_D, _VBLK), jnp.float32),
        pltpu.VMEM((_D, _VBLK), jnp.float32),
        pltpu.VMEM((_VBLK // 2, 2 * _D), jnp.float32),
        pltpu.VMEM((_VBLK // 2, 2 * _D), jnp.float32),
    ]
    + [pltpu.SemaphoreType.DMA for _ in range(4)],
    compiler_params=_params,
)(_stage_body)

_gather_call = functools.partial(
    pl.kernel,
    mesh=_mesh,
    out_type=jax.ShapeDtypeStruct((_SEQ, _D, _B), jnp.float32),
    scratch_types=[
        pltpu.VMEM((_B,), jnp.int32),
        pltpu.VMEM((_B,), jnp.int32),
        pltpu.VMEM((_SEQ * _D,), jnp.float32),
        pltpu.VMEM((_D, _B), jnp.float32),
        pltpu.VMEM((_SUB, 2 * _D), jnp.float32),
        pltpu.VMEM((_SUB, 2 * _D), jnp.float32),
    ]
    + [pltpu.SemaphoreType.DMA for _ in range(3)],
    compiler_params=_params,
)(_gather_body)


@jax.jit
def kernel(x, table):
    xt = x.astype(jnp.int32).T
    tt = table.T
    tail = table[_NBLK_FULL * _VBLK:].reshape(_BLK_REM // 2, 2 * _D)
    stg = _stage_call(tt, tail)
    out = _gather_call(stg, xt, _PE_CONST)
    return out.transpose(2, 0, 1)
